# baseline (speedup 1.0000x reference)
import functools

import jax
import jax.numpy as jnp
from jax import lax
from jax.experimental import pallas as pl
from jax.experimental.pallas import tpu as pltpu
from jax.experimental.pallas import tpu_sc as plsc

_BATCH = 16384
_FEAT = 64
_NC = 2
_NS = 16
_NW = _NC * _NS
_BPW = _BATCH // _NW
_LANES = 16


@functools.partial(
    pl.kernel,
    mesh=plsc.VectorSubcoreMesh(core_axis_name="c", subcore_axis_name="s"),
    out_type=jax.ShapeDtypeStruct((_NW, _LANES), jnp.float32),
    scratch_types=[
        pltpu.VMEM((_BPW,), jnp.int32),
        pltpu.VMEM((_BPW, _FEAT), jnp.float32),
        pltpu.VMEM((_BPW, _FEAT), jnp.float32),
        pltpu.VMEM((_LANES,), jnp.float32),
        pltpu.SemaphoreType.DMA,
    ],
    compiler_params=pltpu.CompilerParams(use_tc_tiling_on_sc=False),
)
def _center_loss_sc(feat_hbm, lab_hbm, cent_hbm, out_hbm,
                    idx_v, cen_v, feat_v, acc_v, sem):
    wid = lax.axis_index("s") * _NC + lax.axis_index("c")
    base = wid * _BPW

    pltpu.sync_copy(lab_hbm.at[pl.ds(base, _BPW)], idx_v)
    gather = pltpu.async_copy(cent_hbm.at[idx_v], cen_v, sem)
    pltpu.sync_copy(feat_hbm.at[pl.ds(base, _BPW)], feat_v)
    gather.wait()

    def body(r, acc):
        for j in range(_FEAT // _LANES):
            f = feat_v[r, pl.ds(j * _LANES, _LANES)]
            c = cen_v[r, pl.ds(j * _LANES, _LANES)]
            d = f - c
            acc = acc + d * d
        return acc

    acc = lax.fori_loop(0, _BPW, body, jnp.zeros((_LANES,), jnp.float32))
    acc_v[...] = acc * (0.5 / _BATCH)
    pltpu.sync_copy(acc_v, out_hbm.at[wid])


def kernel(features, labels, centers):
    partials = _center_loss_sc(features, labels.astype(jnp.int32), centers)
    return jnp.sum(partials)


# speedup vs baseline: 1.3161x; 1.3161x over previous
import functools

import jax
import jax.numpy as jnp
from jax import lax
from jax.experimental import pallas as pl
from jax.experimental.pallas import tpu as pltpu
from jax.experimental.pallas import tpu_sc as plsc

_BATCH = 16384
_FEAT = 64
_NC = 2
_NS = 16
_NW = _NC * _NS
_BPW = _BATCH // _NW
_CHUNK = 256
_LANES = 16


@functools.partial(
    pl.kernel,
    mesh=plsc.VectorSubcoreMesh(core_axis_name="c", subcore_axis_name="s"),
    out_type=jax.ShapeDtypeStruct((_NW, _LANES), jnp.float32),
    scratch_types=[
        pltpu.VMEM((_BPW,), jnp.int32),
        pltpu.VMEM((_CHUNK, _FEAT), jnp.float32),
        pltpu.VMEM((_BPW, _FEAT), jnp.float32),
        pltpu.VMEM((_LANES,), jnp.float32),
        pltpu.SemaphoreType.DMA,
        pltpu.SemaphoreType.DMA,
    ],
)
def _center_loss_sc(feat_hbm, lab_hbm, cent_hbm, out_hbm,
                    idx_v, cen_v, feat_v, acc_v, gsem, fsem):
    wid = lax.axis_index("s") * _NC + lax.axis_index("c")
    base = wid * _BPW

    pltpu.sync_copy(lab_hbm.at[pl.ds(base, _BPW)], idx_v)
    fcopy = pltpu.async_copy(feat_hbm.at[pl.ds(base, _BPW)], feat_v, fsem)

    def run_chunk(c, acc):
        def enqueue(b, _):
            lab_vec = idx_v[pl.ds(c * _CHUNK + b * _LANES, _LANES)]
            for j in range(_LANES):
                r = lab_vec[j]
                pltpu.async_copy(
                    cent_hbm.at[pl.ds(r, 1), :],
                    cen_v.at[pl.ds(b * _LANES + j, 1), :], gsem)
            return 0

        lax.fori_loop(0, _CHUNK // _LANES, enqueue, 0)
        pltpu.make_async_copy(feat_hbm.at[pl.ds(0, _CHUNK)], cen_v,
                              gsem).wait()

        def body(r, a):
            for j in range(_FEAT // _LANES):
                f = feat_v[c * _CHUNK + r, pl.ds(j * _LANES, _LANES)]
                cv = cen_v[r, pl.ds(j * _LANES, _LANES)]
                d = f - cv
                a = a + d * d
            return a

        return lax.fori_loop(0, _CHUNK, body, acc)

    acc = jnp.zeros((_LANES,), jnp.float32)
    fcopy.wait()
    acc = lax.fori_loop(0, _BPW // _CHUNK, run_chunk, acc)
    acc_v[...] = acc * (0.5 / _BATCH)
    pltpu.sync_copy(acc_v, out_hbm.at[wid])


def kernel(features, labels, centers):
    partials = _center_loss_sc(features, labels.astype(jnp.int32), centers)
    return jnp.sum(partials)


# speedup vs baseline: 1.9409x; 1.4748x over previous
import functools

import jax
import jax.numpy as jnp
from jax import lax
from jax.experimental import pallas as pl
from jax.experimental.pallas import tpu as pltpu
from jax.experimental.pallas import tpu_sc as plsc

_BATCH = 16384
_NCLASS = 100000
_FEAT = 64
_NC = 2
_NS = 16
_NW = _NC * _NS
_FPW = _FEAT // _NW
_LANES = 16

_CW = 12544
_NK = 8
_K7OFF = _NCLASS - 32 - _CW
_SEG = 2048
_NSEG = _BATCH // _SEG
_MRING = 2


@functools.partial(
    pl.kernel,
    mesh=plsc.VectorSubcoreMesh(core_axis_name="c", subcore_axis_name="s"),
    out_type=jax.ShapeDtypeStruct((_NW, _LANES), jnp.float32),
    scratch_types=[
        pltpu.VMEM((_NK, _CW), jnp.float32),
        pltpu.VMEM((_NSEG, _SEG), jnp.float32),
        pltpu.VMEM((_MRING * _SEG,), jnp.int32),
        pltpu.VMEM((_FEAT, 32), jnp.float32),
        pltpu.VMEM((_LANES,), jnp.float32),
        pltpu.SemaphoreType.DMA,
        pltpu.SemaphoreType.DMA,
        [pltpu.SemaphoreType.DMA for _ in range(_MRING)],
    ],
    compiler_params=pltpu.CompilerParams(needs_layout_passes=False),
)
def _center_loss_sc(feat_hbm, m_hbm, cent_hbm, out_hbm,
                    crow_v, fv, m_v, tail_v, acc_v, csem, fsem, msems):
    wid = lax.axis_index("s") * _NC + lax.axis_index("c")
    z = lax.shift_right_logical(wid, 5)

    pltpu.sync_copy(cent_hbm.at[pl.ds(0, _FEAT), pl.ds(_NCLASS - 32, 32)],
                    tail_v)

    acc = jnp.zeros((_LANES,), jnp.float32)
    for p in range(_FPW):
        f = wid * _FPW + p
        for k in range(_NK):
            off = k * _CW if k < _NK - 1 else _K7OFF
            pltpu.async_copy(
                cent_hbm.at[pl.ds(f, 1), pl.ds(off, _CW)],
                crow_v.at[pl.ds(k + z, 1), :], csem)
        for s in range(_NSEG):
            pltpu.async_copy(
                feat_hbm.at[pl.ds(f, 1), pl.ds(s * _SEG, _SEG)],
                fv.at[pl.ds(s + z, 1), :], fsem)
        for s in range(_MRING):
            pltpu.async_copy(m_hbm.at[pl.ds(s * _SEG, _SEG)],
                             m_v.at[pl.ds(s * _SEG, _SEG)], msems[s])
        pltpu.make_async_copy(
            cent_hbm.at[pl.ds(0, _NK), pl.ds(0, _CW)], crow_v, csem).wait()
        pltpu.make_async_copy(
            feat_hbm.at[pl.ds(0, _NSEG), pl.ds(0, _SEG)], fv, fsem).wait()
        k7 = _NK - 1 + z
        crow_v[k7, pl.ds(0, _LANES)] = tail_v[f, pl.ds(0, _LANES)]
        crow_v[k7, pl.ds(_LANES, _LANES)] = tail_v[f, pl.ds(_LANES, _LANES)]

        for s in range(_NSEG):
            slot = s % _MRING
            pltpu.make_async_copy(
                m_hbm.at[pl.ds(0, _SEG)],
                m_v.at[pl.ds(slot * _SEG, _SEG)], msems[slot]).wait()

            def seg_body(vl, a, s=s, slot=slot):
                m16 = m_v[pl.ds(slot * _SEG + vl * _LANES, _LANES)]
                k = lax.shift_right_logical(m16, 14)
                j = m16 & 16383
                c = plsc.load_gather(crow_v, [k, j])
                fvv = fv[s + z, pl.ds(vl * _LANES, _LANES)]
                d = fvv - c
                return a + d * d

            acc = lax.fori_loop(0, _SEG // _LANES, seg_body, acc)
            if s + _MRING < _NSEG:
                nxt = s + _MRING
                pltpu.async_copy(m_hbm.at[pl.ds(nxt * _SEG, _SEG)],
                                 m_v.at[pl.ds(slot * _SEG, _SEG)],
                                 msems[slot])

    acc_v[...] = acc * (0.5 / _BATCH)
    pltpu.sync_copy(acc_v, out_hbm.at[wid])


def kernel(features, labels, centers):
    labels = labels.astype(jnp.int32)
    k = jnp.minimum(labels // _CW, _NK - 1)
    j = jnp.where(labels < (_NK - 1) * _CW, labels - k * _CW,
                  jnp.where(labels < _NCLASS - 32, labels - _K7OFF,
                            labels - (_NCLASS - 32)))
    m = (k << 14) | j
    partials = _center_loss_sc(features.T, m, centers.T)
    return jnp.sum(partials)


# speedup vs baseline: 1.9581x; 1.0088x over previous
import functools

import jax
import jax.numpy as jnp
from jax import lax
from jax.experimental import pallas as pl
from jax.experimental.pallas import tpu as pltpu
from jax.experimental.pallas import tpu_sc as plsc

_BATCH = 16384
_NCLASS = 100000
_FEAT = 64
_NC = 2
_NS = 16
_NW = _NC * _NS
_FPW = _FEAT // _NW
_LANES = 16

_CW = 12544
_NK = 8
_K7OFF = _NCLASS - 32 - _CW
_SEG = 2048
_NSEG = _BATCH // _SEG
_MRING = 2


@functools.partial(
    pl.kernel,
    mesh=plsc.VectorSubcoreMesh(core_axis_name="c", subcore_axis_name="s"),
    out_type=jax.ShapeDtypeStruct((_NW, _LANES), jnp.float32),
    scratch_types=[
        pltpu.VMEM((_NK, _CW), jnp.float32),
        pltpu.VMEM((_NSEG, _SEG), jnp.float32),
        pltpu.VMEM((_MRING * _SEG,), jnp.int32),
        pltpu.VMEM((_FEAT, 32), jnp.float32),
        pltpu.VMEM((_LANES,), jnp.float32),
        pltpu.SemaphoreType.DMA,
        pltpu.SemaphoreType.DMA,
        [pltpu.SemaphoreType.DMA for _ in range(_MRING)],
    ],
    compiler_params=pltpu.CompilerParams(needs_layout_passes=False),
)
def _center_loss_sc(feat_hbm, m_hbm, cent_hbm, out_hbm,
                    crow_v, fv, m_v, tail_v, acc_v, csem, fsem, msems):
    wid = lax.axis_index("s") * _NC + lax.axis_index("c")
    z = lax.shift_right_logical(wid, 5)

    pltpu.sync_copy(cent_hbm.at[pl.ds(0, _FEAT), pl.ds(_NCLASS - 32, 32)],
                    tail_v)

    acc = jnp.zeros((_LANES,), jnp.float32)
    for p in range(_FPW):
        f = wid * _FPW + p
        for k in range(_NK):
            off = k * _CW if k < _NK - 1 else _K7OFF
            pltpu.async_copy(
                cent_hbm.at[pl.ds(f, 1), pl.ds(off, _CW)],
                crow_v.at[pl.ds(k + z, 1), :], csem)
        for s in range(_NSEG):
            pltpu.async_copy(
                feat_hbm.at[pl.ds(f, 1), pl.ds(s * _SEG, _SEG)],
                fv.at[pl.ds(s + z, 1), :], fsem)
        for s in range(_MRING):
            pltpu.async_copy(m_hbm.at[pl.ds(s * _SEG, _SEG)],
                             m_v.at[pl.ds(s * _SEG, _SEG)], msems[s])
        pltpu.make_async_copy(
            cent_hbm.at[pl.ds(0, _NK), pl.ds(0, _CW)], crow_v, csem).wait()
        pltpu.make_async_copy(
            feat_hbm.at[pl.ds(0, _NSEG), pl.ds(0, _SEG)], fv, fsem).wait()
        k7 = _NK - 1 + z
        crow_v[k7, pl.ds(0, _LANES)] = tail_v[f, pl.ds(0, _LANES)]
        crow_v[k7, pl.ds(_LANES, _LANES)] = tail_v[f, pl.ds(_LANES, _LANES)]

        for s in range(_NSEG):
            slot = s % _MRING
            pltpu.make_async_copy(
                m_hbm.at[pl.ds(0, _SEG)],
                m_v.at[pl.ds(slot * _SEG, _SEG)], msems[slot]).wait()

            def seg_body(vl, a, s=s, slot=slot):
                m16 = m_v[pl.ds(slot * _SEG + vl * _LANES, _LANES)]
                k = lax.shift_right_logical(m16, 14)
                j = m16 & 16383
                c = plsc.load_gather(crow_v, [k, j])
                fvv = fv[s + z, pl.ds(vl * _LANES, _LANES)]
                d = fvv - c
                return a + d * d

            acc = lax.fori_loop(0, _SEG // _LANES, seg_body, acc, unroll=4)
            if s + _MRING < _NSEG:
                nxt = s + _MRING
                pltpu.async_copy(m_hbm.at[pl.ds(nxt * _SEG, _SEG)],
                                 m_v.at[pl.ds(slot * _SEG, _SEG)],
                                 msems[slot])

    acc_v[...] = acc * (0.5 / _BATCH)
    pltpu.sync_copy(acc_v, out_hbm.at[wid])


def kernel(features, labels, centers):
    labels = labels.astype(jnp.int32)
    k = jnp.minimum(labels // _CW, _NK - 1)
    j = jnp.where(labels < (_NK - 1) * _CW, labels - k * _CW,
                  jnp.where(labels < _NCLASS - 32, labels - _K7OFF,
                            labels - (_NCLASS - 32)))
    m = (k << 14) | j
    partials = _center_loss_sc(features.T, m, centers.T)
    return jnp.sum(partials)


# speedup vs baseline: 2.0933x; 1.0690x over previous
import functools

import jax
import jax.numpy as jnp
from jax import lax
from jax.experimental import pallas as pl
from jax.experimental.pallas import tpu as pltpu
from jax.experimental.pallas import tpu_sc as plsc

_BATCH = 16384
_NCLASS = 100000
_FEAT = 64
_NC = 2
_NS = 16
_NW = _NC * _NS
_FPW = _FEAT // _NW
_LANES = 16

_CW = 12544
_NK = 8
_K7OFF = _NCLASS - 32 - _CW
_SEG = 2048
_NSEG = _BATCH // _SEG
_MRING = 2


@functools.partial(
    pl.kernel,
    mesh=plsc.VectorSubcoreMesh(core_axis_name="c", subcore_axis_name="s"),
    out_type=jax.ShapeDtypeStruct((_NW, _LANES), jnp.float32),
    scratch_types=[
        pltpu.VMEM((_NK, _CW), jnp.float32),
        pltpu.VMEM((_NSEG, _SEG), jnp.float32),
        pltpu.VMEM((_MRING * _SEG,), jnp.int32),
        pltpu.VMEM((_FEAT, 32), jnp.float32),
        pltpu.VMEM((_LANES,), jnp.float32),
        pltpu.SemaphoreType.DMA,
        pltpu.SemaphoreType.DMA,
        pltpu.SemaphoreType.DMA((_MRING,)),
    ],
    compiler_params=pltpu.CompilerParams(needs_layout_passes=False),
)
def _center_loss_sc(feat_hbm, m_hbm, cent_hbm, out_hbm,
                    crow_v, fv, m_v, tail_v, acc_v, csem, fsem, msems):
    wid = lax.axis_index("s") * _NC + lax.axis_index("c")
    z = lax.shift_right_logical(wid, 5)

    pltpu.sync_copy(cent_hbm.at[pl.ds(0, _FEAT), pl.ds(_NCLASS - 32, 32)],
                    tail_v)

    acc = jnp.zeros((_LANES,), jnp.float32)
    for p in range(_FPW):
        f = wid * _FPW + p

        def stage_crow(k, _, f=f):
            off = jnp.where(k < _NK - 1, k * _CW, _K7OFF)
            pltpu.async_copy(
                cent_hbm.at[pl.ds(f, 1), pl.ds(off, _CW)],
                crow_v.at[pl.ds(k + z, 1), :], csem)
            return 0

        lax.fori_loop(0, _NK, stage_crow, 0)

        def stage_fv(s, _, f=f):
            pltpu.async_copy(
                feat_hbm.at[pl.ds(f, 1), pl.ds(s * _SEG, _SEG)],
                fv.at[pl.ds(s + z, 1), :], fsem)
            return 0

        lax.fori_loop(0, _NSEG, stage_fv, 0)

        def prefetch_m(s, _):
            pltpu.async_copy(m_hbm.at[pl.ds(s * _SEG, _SEG)],
                             m_v.at[pl.ds(s * _SEG, _SEG)], msems.at[s])
            return 0

        lax.fori_loop(0, _MRING, prefetch_m, 0)

        pltpu.make_async_copy(
            cent_hbm.at[pl.ds(0, _NK), pl.ds(0, _CW)], crow_v, csem).wait()
        pltpu.make_async_copy(
            feat_hbm.at[pl.ds(0, _NSEG), pl.ds(0, _SEG)], fv, fsem).wait()
        k7 = _NK - 1 + z
        crow_v[k7, pl.ds(0, _LANES)] = tail_v[f, pl.ds(0, _LANES)]
        crow_v[k7, pl.ds(_LANES, _LANES)] = tail_v[f, pl.ds(_LANES, _LANES)]

        def seg_fn(s, a):
            slot = s % _MRING
            pltpu.make_async_copy(
                m_hbm.at[pl.ds(0, _SEG)],
                m_v.at[pl.ds(slot * _SEG, _SEG)], msems.at[slot]).wait()

            def seg_body(vl, aa, s=s, slot=slot):
                m16 = m_v[pl.ds(slot * _SEG + vl * _LANES, _LANES)]
                k = lax.shift_right_logical(m16, 14)
                j = m16 & 16383
                c = plsc.load_gather(crow_v, [k, j])
                fvv = fv[s + z, pl.ds(vl * _LANES, _LANES)]
                d = fvv - c
                return aa + d * d

            a = lax.fori_loop(0, _SEG // _LANES, seg_body, a, unroll=2)

            @pl.when(s + _MRING < _NSEG)
            def _():
                nxt = s + _MRING
                pltpu.async_copy(m_hbm.at[pl.ds(nxt * _SEG, _SEG)],
                                 m_v.at[pl.ds(slot * _SEG, _SEG)],
                                 msems.at[slot])

            return a

        acc = lax.fori_loop(0, _NSEG, seg_fn, acc)

    acc_v[...] = acc * (0.5 / _BATCH)
    pltpu.sync_copy(acc_v, out_hbm.at[wid])


def kernel(features, labels, centers):
    labels = labels.astype(jnp.int32)
    k = jnp.minimum(labels // _CW, _NK - 1)
    j = jnp.where(labels < (_NK - 1) * _CW, labels - k * _CW,
                  jnp.where(labels < _NCLASS - 32, labels - _K7OFF,
                            labels - (_NCLASS - 32)))
    m = (k << 14) | j
    partials = _center_loss_sc(features.T, m, centers.T)
    return jnp.sum(partials)


# speedup vs baseline: 2.0996x; 1.0030x over previous
import functools

import jax
import jax.numpy as jnp
from jax import lax
from jax.experimental import pallas as pl
from jax.experimental.pallas import tpu as pltpu
from jax.experimental.pallas import tpu_sc as plsc

_BATCH = 16384
_NCLASS = 100000
_FEAT = 64
_NC = 2
_NS = 16
_NW = _NC * _NS
_FPW = _FEAT // _NW
_LANES = 16

_CW = 12544
_NK = 8
_K7OFF = _NCLASS - 32 - _CW
_SEG = 2048
_NSEG = _BATCH // _SEG
_MRING = 2


@functools.partial(
    pl.kernel,
    mesh=plsc.VectorSubcoreMesh(core_axis_name="c", subcore_axis_name="s"),
    out_type=jax.ShapeDtypeStruct((_NW, _LANES), jnp.float32),
    scratch_types=[
        pltpu.VMEM((_NK, _CW), jnp.float32),
        pltpu.VMEM((_NSEG, _SEG), jnp.float32),
        pltpu.VMEM((_MRING * _SEG,), jnp.int32),
        pltpu.VMEM((_FEAT, 32), jnp.float32),
        pltpu.VMEM((_LANES,), jnp.float32),
        pltpu.SemaphoreType.DMA,
        pltpu.SemaphoreType.DMA,
        pltpu.SemaphoreType.DMA((_MRING,)),
        pltpu.SemaphoreType.DMA,
    ],
    compiler_params=pltpu.CompilerParams(needs_layout_passes=False),
)
def _center_loss_sc(feat_hbm, m_hbm, cent_hbm, out_hbm,
                    crow_v, fv, m_v, tail_v, acc_v, csem, fsem, msems,
                    tsem):
    wid = lax.axis_index("s") * _NC + lax.axis_index("c")
    z = lax.shift_right_logical(wid, 5)

    tail_copy = pltpu.async_copy(
        cent_hbm.at[pl.ds(0, _FEAT), pl.ds(_NCLASS - 32, 32)], tail_v, tsem)

    def pass_fn(p, acc):
        f = wid * _FPW + p

        def stage_crow(k, _, f=f):
            off = jnp.where(k < _NK - 1, k * _CW, _K7OFF)
            pltpu.async_copy(
                cent_hbm.at[pl.ds(f, 1), pl.ds(off, _CW)],
                crow_v.at[pl.ds(k + z, 1), :], csem)
            return 0

        lax.fori_loop(0, _NK, stage_crow, 0)

        def stage_fv(s, _, f=f):
            pltpu.async_copy(
                feat_hbm.at[pl.ds(f, 1), pl.ds(s * _SEG, _SEG)],
                fv.at[pl.ds(s + z, 1), :], fsem)
            return 0

        lax.fori_loop(0, _NSEG, stage_fv, 0)

        def prefetch_m(s, _):
            pltpu.async_copy(m_hbm.at[pl.ds(s * _SEG, _SEG)],
                             m_v.at[pl.ds(s * _SEG, _SEG)], msems.at[s])
            return 0

        lax.fori_loop(0, _MRING, prefetch_m, 0)

        pltpu.make_async_copy(
            cent_hbm.at[pl.ds(0, _NK), pl.ds(0, _CW)], crow_v, csem).wait()
        pltpu.make_async_copy(
            feat_hbm.at[pl.ds(0, _NSEG), pl.ds(0, _SEG)], fv, fsem).wait()

        @pl.when(p == 0)
        def _():
            tail_copy.wait()

        k7 = _NK - 1 + z
        crow_v[k7, pl.ds(0, _LANES)] = tail_v[f, pl.ds(0, _LANES)]
        crow_v[k7, pl.ds(_LANES, _LANES)] = tail_v[f, pl.ds(_LANES, _LANES)]

        def seg_fn(s, a):
            slot = s % _MRING
            pltpu.make_async_copy(
                m_hbm.at[pl.ds(0, _SEG)],
                m_v.at[pl.ds(slot * _SEG, _SEG)], msems.at[slot]).wait()

            def seg_body(vl, aa, s=s, slot=slot):
                m16 = m_v[pl.ds(slot * _SEG + vl * _LANES, _LANES)]
                k = lax.shift_right_logical(m16, 14)
                j = m16 & 16383
                c = plsc.load_gather(crow_v, [k, j])
                fvv = fv[s + z, pl.ds(vl * _LANES, _LANES)]
                d = fvv - c
                return aa + d * d

            a = lax.fori_loop(0, _SEG // _LANES, seg_body, a, unroll=2)

            @pl.when(s + _MRING < _NSEG)
            def _():
                nxt = s + _MRING
                pltpu.async_copy(m_hbm.at[pl.ds(nxt * _SEG, _SEG)],
                                 m_v.at[pl.ds(slot * _SEG, _SEG)],
                                 msems.at[slot])

            return a

        return lax.fori_loop(0, _NSEG, seg_fn, acc)

    acc = lax.fori_loop(0, _FPW, pass_fn,
                        jnp.zeros((_LANES,), jnp.float32))
    acc_v[...] = acc * (0.5 / _BATCH)
    pltpu.sync_copy(acc_v, out_hbm.at[wid])


def kernel(features, labels, centers):
    labels = labels.astype(jnp.int32)
    k = jnp.minimum(labels // _CW, _NK - 1)
    j = jnp.where(labels < (_NK - 1) * _CW, labels - k * _CW,
                  jnp.where(labels < _NCLASS - 32, labels - _K7OFF,
                            labels - (_NCLASS - 32)))
    m = (k << 14) | j
    partials = _center_loss_sc(features.T, m, centers.T)
    return jnp.sum(partials)


# speedup vs baseline: 2.1257x; 1.0124x over previous
import functools

import jax
import jax.numpy as jnp
from jax import lax
from jax.experimental import pallas as pl
from jax.experimental.pallas import tpu as pltpu
from jax.experimental.pallas import tpu_sc as plsc

_BATCH = 16384
_NCLASS = 100000
_FEAT = 64
_NC = 2
_NS = 16
_NW = _NC * _NS
_FPW = _FEAT // _NW
_LANES = 16

_CW = 12544
_NK = 8
_K7OFF = _NCLASS - 32 - _CW
_SEG = 2048
_NSEG = _BATCH // _SEG
_MRING = 2


@functools.partial(
    pl.kernel,
    mesh=plsc.VectorSubcoreMesh(core_axis_name="c", subcore_axis_name="s"),
    out_type=jax.ShapeDtypeStruct((_NW, _LANES), jnp.float32),
    scratch_types=[
        pltpu.VMEM((_NK, _CW), jnp.float32),
        pltpu.VMEM((_NSEG, _SEG), jnp.float32),
        pltpu.VMEM((_MRING * _SEG,), jnp.int32),
        pltpu.VMEM((_FEAT, 32), jnp.float32),
        pltpu.VMEM((_LANES,), jnp.float32),
        pltpu.SemaphoreType.DMA,
        pltpu.SemaphoreType.DMA,
        pltpu.SemaphoreType.DMA((_MRING,)),
        pltpu.SemaphoreType.DMA,
    ],
    compiler_params=pltpu.CompilerParams(needs_layout_passes=False),
)
def _center_loss_sc(feat_hbm, m_hbm, cent_hbm, out_hbm,
                    crow_v, fv, m_v, tail_v, acc_v, csem, fsem, msems,
                    tsem):
    wid = lax.axis_index("s") * _NC + lax.axis_index("c")
    z = lax.shift_right_logical(wid, 5)

    tail_copy = pltpu.async_copy(
        cent_hbm.at[pl.ds(0, _FEAT), pl.ds(_NCLASS - 32, 32)], tail_v, tsem)

    def pass_fn(p, acc):
        f = wid * _FPW + p

        def stage_crow(k, _, f=f):
            off = jnp.where(k < _NK - 1, k * _CW, _K7OFF)
            pltpu.async_copy(
                cent_hbm.at[pl.ds(f, 1), pl.ds(off, _CW)],
                crow_v.at[pl.ds(k + z, 1), :], csem)
            return 0

        lax.fori_loop(0, _NK, stage_crow, 0)

        def stage_fv(s, _, f=f):
            pltpu.async_copy(
                feat_hbm.at[pl.ds(f, 1), pl.ds(s * _SEG, _SEG)],
                fv.at[pl.ds(s + z, 1), :], fsem)
            return 0

        lax.fori_loop(0, _NSEG, stage_fv, 0)

        def prefetch_m(s, _):
            pltpu.async_copy(m_hbm.at[pl.ds(s * _SEG, _SEG)],
                             m_v.at[pl.ds(s * _SEG, _SEG)], msems.at[s])
            return 0

        lax.fori_loop(0, _MRING, prefetch_m, 0)

        pltpu.make_async_copy(
            cent_hbm.at[pl.ds(0, _NK), pl.ds(0, _CW)], crow_v, csem).wait()
        pltpu.make_async_copy(
            feat_hbm.at[pl.ds(0, _NSEG), pl.ds(0, _SEG)], fv, fsem).wait()

        @pl.when(p == 0)
        def _():
            tail_copy.wait()

        k7 = _NK - 1 + z
        crow_v[k7, pl.ds(0, _LANES)] = tail_v[f, pl.ds(0, _LANES)]
        crow_v[k7, pl.ds(_LANES, _LANES)] = tail_v[f, pl.ds(_LANES, _LANES)]

        def seg_fn(s, a):
            slot = s % _MRING
            pltpu.make_async_copy(
                m_hbm.at[pl.ds(0, _SEG)],
                m_v.at[pl.ds(slot * _SEG, _SEG)], msems.at[slot]).wait()

            def seg_body(vl, aa, s=s, slot=slot):
                m16 = m_v[pl.ds(slot * _SEG + vl * _LANES, _LANES)]
                k = lax.shift_right_logical(m16, 14)
                j = m16 & 16383
                c = plsc.load_gather(crow_v, [k, j])
                fvv = fv[s + z, pl.ds(vl * _LANES, _LANES)]
                d = fvv - c
                return aa + d * d

            a = lax.fori_loop(0, _SEG // _LANES, seg_body, a, unroll=4)

            @pl.when(s + _MRING < _NSEG)
            def _():
                nxt = s + _MRING
                pltpu.async_copy(m_hbm.at[pl.ds(nxt * _SEG, _SEG)],
                                 m_v.at[pl.ds(slot * _SEG, _SEG)],
                                 msems.at[slot])

            return a

        return lax.fori_loop(0, _NSEG, seg_fn, acc)

    acc = lax.fori_loop(0, _FPW, pass_fn,
                        jnp.zeros((_LANES,), jnp.float32))
    acc_v[...] = acc * (0.5 / _BATCH)
    pltpu.sync_copy(acc_v, out_hbm.at[wid])


def kernel(features, labels, centers):
    labels = labels.astype(jnp.int32)
    k = jnp.minimum(labels // _CW, _NK - 1)
    j = jnp.where(labels < (_NK - 1) * _CW, labels - k * _CW,
                  jnp.where(labels < _NCLASS - 32, labels - _K7OFF,
                            labels - (_NCLASS - 32)))
    m = (k << 14) | j
    partials = _center_loss_sc(features.T, m, centers.T)
    return jnp.sum(partials)
